# baseline (speedup 1.0000x reference)
import functools
import math

import jax
import jax.numpy as jnp
from jax import lax
from jax.experimental import pallas as pl
from jax.experimental.pallas import tpu as pltpu
from jax.experimental.pallas import tpu_sc as plsc

MIN_NORM = 1e-15
RADIUS = 0.1
B = 4096
K = 50
D = 64
NG = 7
PW = NG * 16

NC = 2
NS = 16
NW = NC * NS
RPW = B // NW

WP = 500000


def _bitrev4(j):
    return int(f"{j:04b}"[::-1], 2)


def _sc_kernel(idx_hbm, pp_hbm, pv_hbm, w_hbm, pk_hbm,
               idx_v, pp_v, pv_v, buf_v, pk_v, sems):
    wid = lax.axis_index("s") * NC + lax.axis_index("c")
    pltpu.sync_copy(idx_hbm.at[wid], idx_v.at[pl.ds(0, RPW)])
    pltpu.sync_copy(idx_hbm.at[wid, pl.ds(0, 1)], idx_v.at[pl.ds(RPW, 1)])
    pltpu.sync_copy(pp_hbm.at[wid], pp_v)
    pltpu.sync_copy(pv_hbm.at[wid], pv_v)

    lane = lax.iota(jnp.int32, 16)
    zero = jnp.zeros((16,), jnp.float32)
    one = jnp.full((16,), 1.0, jnp.float32)
    dnums = lax.GatherDimensionNumbers(
        offset_dims=(), collapsed_slice_dims=(0,), start_index_map=(0,))

    def _take16(v, idx):
        return lax.gather(v, idx[:, None], dnums, (1,),
                          mode=lax.GatherScatterMode.PROMISE_IN_BOUNDS)

    masks = {s: (lane & s) == 0 for s in (8, 4, 2, 1)}
    perms = {s: lane ^ s for s in (8, 4, 2, 1)}

    def merge(x, y, s):
        return jnp.where(masks[s], x + _take16(x, perms[s]),
                         y + _take16(y, perms[s]))

    def start(c):
        slot = lax.rem(c, 2)
        return pltpu.async_copy(w_hbm.at[idx_v.at[c]], buf_v.at[slot],
                                sems.at[slot])

    def wait(c):
        slot = lax.rem(c, 2)
        pltpu.make_async_copy(w_hbm.at[idx_v.at[0]], buf_v.at[slot],
                              sems.at[slot]).wait()

    start(0)

    @pl.loop(0, RPW)
    def body(r):
        start(r + 1)
        wait(r)
        slot = lax.rem(r, 2)
        pp16 = pp_v[r]
        ppn = one - pp16
        p = [buf_v[slot, 0, pl.ds(t * 16, 16)] * ppn +
             buf_v[slot, 0, pl.ds(64 + t * 16, 16)] * pp16 for t in range(4)]
        for kb in range(NG):
            des_lo = []
            des_hi = []
            for i in range(8):
                k = kb * 8 + i
                if k < K:
                    clo = [buf_v[slot, k, pl.ds(t * 16, 16)]
                           for t in range(4)]
                    chi = [buf_v[slot, k, pl.ds(64 + t * 16, 16)]
                           for t in range(4)]
                    dlo = p[0] * clo[0]
                    nlo = clo[0] * clo[0]
                    dhi = p[0] * chi[0]
                    nhi = chi[0] * chi[0]
                    for t in range(1, 4):
                        dlo = dlo + p[t] * clo[t]
                        nlo = nlo + clo[t] * clo[t]
                        dhi = dhi + p[t] * chi[t]
                        nhi = nhi + chi[t] * chi[t]
                    des_lo.append((dlo, nlo))
                    des_hi.append((dhi, nhi))
                else:
                    des_lo.append((zero, zero))
                    des_hi.append((zero, zero))
            packed = []
            for des in (des_lo, des_hi):
                a = [None] * 16
                for j in range(16):
                    a[_bitrev4(j)] = des[j][0] if j < 8 else des[j - 8][1]
                lvl = a
                for s in (8, 4, 2, 1):
                    lvl = [merge(lvl[2 * i], lvl[2 * i + 1], s)
                           for i in range(len(lvl) // 2)]
                packed.append(lvl[0])
            pvv = pv_v[r, pl.ds(kb * 16, 16)]
            pk_v[r, pl.ds(kb * 16, 16)] = (packed[0] * (one - pvv) +
                                           packed[1] * pvv)

    wait(RPW)
    pltpu.sync_copy(pk_v, pk_hbm.at[pl.ds(wid * RPW, RPW)])


@functools.cache
def _sc_call():
    return functools.partial(
        pl.kernel,
        mesh=plsc.VectorSubcoreMesh(core_axis_name="c", subcore_axis_name="s"),
        out_type=jax.ShapeDtypeStruct((B, PW), jnp.float32),
        scratch_types=[
            pltpu.VMEM((RPW + 1, K), jnp.int32),
            pltpu.VMEM((RPW, 16), jnp.float32),
            pltpu.VMEM((RPW, PW), jnp.float32),
            pltpu.VMEM((2, K, 128), jnp.float32),
            pltpu.VMEM((RPW, PW), jnp.float32),
            pltpu.SemaphoreType.DMA((2,)),
        ],
        compiler_params=pltpu.CompilerParams(use_tc_tiling_on_sc=True),
    )(_sc_kernel)


RB = 4000


def _repack_kernel(w_ref, out_ref):
    out_ref[...] = jnp.concatenate(
        [w_ref[pl.ds(0, RB)], w_ref[pl.ds(RB, RB)]], axis=1)


def _repack(w):
    n = w.shape[0]
    nb = (n // 2) // RB
    return pl.pallas_call(
        _repack_kernel,
        grid=(nb,),
        in_specs=[pl.BlockSpec((2 * RB, D), lambda i: (i, 0))],
        out_specs=pl.BlockSpec((RB, 2 * D), lambda i: (i, 0)),
        out_shape=jax.ShapeDtypeStruct((n // 2, 2 * D), jnp.float32),
    )(w)


def _acos(x):
    return jnp.arctan2(jnp.sqrt(jnp.maximum(1.0 - x * x, 0.0)), x)


def _asin(x):
    return jnp.arctan2(x, jnp.sqrt(jnp.maximum(1.0 - x * x, 0.0)))


def _tc_kernel(pk_ref, out_ref):
    pk = pk_ref[...]
    dt = jnp.concatenate(
        [pk[:, kb * 16:kb * 16 + 8] for kb in range(NG)], axis=1)
    n2 = jnp.concatenate(
        [pk[:, kb * 16 + 8:kb * 16 + 16] for kb in range(NG)], axis=1)

    maxnorm = 1.0 - 1e-5
    sinh_cr = math.sinh(RADIUS)

    raw_n = jnp.sqrt(n2)
    n_cl = jnp.maximum(raw_n, MIN_NORM)
    f = jnp.where(n_cl > maxnorm, maxnorm / n_cl, 1.0)
    proj_n = raw_n * f
    f0 = f[:, 0:1]
    dot_p = dt * (f * f0)

    np_a = jnp.maximum(proj_n[:, 0:1], MIN_NORM)
    np2_a = np_a * np_a
    sin_beta = sinh_cr * (1.0 - np2_a) / (2.0 * np_a)
    beta = _asin(jnp.clip(sin_beta, -1.0, 1.0))
    nc_a = jnp.maximum(proj_n, MIN_NORM)
    cos_alpha = dot_p / (np_a * nc_a)
    cos_alpha = jnp.clip(cos_alpha, -1.0 + 1e-7, 1.0 - 1e-7)
    alpha = _acos(cos_alpha)
    denom = nc_a * jnp.sin(beta - alpha) - np_a * sin_beta
    denom = jnp.where(jnp.abs(denom) < 1e-12, -1e-12, denom)
    h = 0.5 * (np2_a - nc_a * nc_a) / denom
    r_x_y = np2_a + h * h + 2.0 * h * np_a * sin_beta
    cos_angle = jnp.where(h < r_x_y,
                          h / r_x_y * jnp.cos(beta),
                          h / r_x_y * jnp.sin(beta))
    angle_ = _acos(jnp.clip(cos_angle, -1.0, 1.0))
    angle = jnp.where(alpha > beta, angle_, alpha)

    np_h = proj_n[:, 0:1]
    np2_h = np_h * np_h
    sin_beta_h = sinh_cr * (1.0 - np2_h) / (2.0 * np_h)
    hx = (1.0 - np2_h) / (2.0 * sin_beta_h)
    cos_theta = (1.0 + np2_h) / (2.0 * jnp.sqrt(hx * hx + np2_h))
    half_ap = jnp.pi / 2.0 - _acos(jnp.clip(cos_theta, -1.0, 1.0))

    out_ref[...] = angle - half_ap


def _tc_call(pk):
    return pl.pallas_call(
        _tc_kernel,
        out_shape=jax.ShapeDtypeStruct((B, NG * 8), jnp.float32),
    )(pk)


def kernel(inputs, weight):
    idx = inputs.astype(jnp.int32)
    blk = idx // (2 * RB)
    rem = idx - blk * (2 * RB)
    hi = rem >= RB
    i2 = (blk * RB + jnp.where(hi, rem - RB, rem)).reshape(NW, RPW, K)
    par = hi.astype(jnp.float32)
    pp = jnp.broadcast_to(par[:, 0:1], (B, 16)).reshape(NW, RPW, 16)
    grp = jnp.pad(par, ((0, 0), (0, NG * 8 - K)))
    pv = jnp.concatenate(
        [grp.reshape(B, NG, 1, 8)] * 2, axis=2).reshape(NW, RPW, PW)
    w2 = _repack(weight)
    pk = _sc_call()(i2, pp, pv, w2)
    out = _tc_call(pk)
    return out[:, 1:K]


# speedup vs baseline: 1.2231x; 1.2231x over previous
import functools
import math

import jax
import jax.numpy as jnp
from jax import lax
from jax.experimental import pallas as pl
from jax.experimental.pallas import tpu as pltpu
from jax.experimental.pallas import tpu_sc as plsc

MIN_NORM = 1e-15
RADIUS = 0.1
B = 4096
K = 50
D = 64
NG = 7
PW = NG * 16

NC = 2
NS = 16
NW = NC * NS
RPW = B // NW

WP = 500000


def _bitrev4(j):
    return int(f"{j:04b}"[::-1], 2)


def _sc_kernel(idx_hbm, pp_hbm, pv_hbm, w_hbm, pk_hbm,
               idx_v, pp_v, pv_v, buf_v, pk_v, sems):
    wid = lax.axis_index("s") * NC + lax.axis_index("c")
    pltpu.sync_copy(idx_hbm.at[wid], idx_v.at[pl.ds(0, RPW)])
    pltpu.sync_copy(idx_hbm.at[wid, pl.ds(0, 1)], idx_v.at[pl.ds(RPW, 1)])
    pltpu.sync_copy(pp_hbm.at[wid], pp_v)
    pltpu.sync_copy(pv_hbm.at[wid], pv_v)

    lane = lax.iota(jnp.int32, 16)
    zero = jnp.zeros((16,), jnp.float32)
    one = jnp.full((16,), 1.0, jnp.float32)
    dnums = lax.GatherDimensionNumbers(
        offset_dims=(), collapsed_slice_dims=(0,), start_index_map=(0,))

    def _take16(v, idx):
        return lax.gather(v, idx[:, None], dnums, (1,),
                          mode=lax.GatherScatterMode.PROMISE_IN_BOUNDS)

    masks = {s: (lane & s) == 0 for s in (8, 4, 2, 1)}
    perms = {s: lane ^ s for s in (8, 4, 2, 1)}

    def merge(x, y, s):
        return jnp.where(masks[s], x + _take16(x, perms[s]),
                         y + _take16(y, perms[s]))

    def start(c):
        slot = lax.rem(c, 2)
        return pltpu.async_copy(w_hbm.at[idx_v.at[c]], buf_v.at[slot],
                                sems.at[slot])

    def wait(c):
        slot = lax.rem(c, 2)
        pltpu.make_async_copy(w_hbm.at[idx_v.at[0]], buf_v.at[slot],
                              sems.at[slot]).wait()

    start(0)

    @pl.loop(0, RPW)
    def body(r):
        start(r + 1)
        wait(r)
        slot = lax.rem(r, 2)
        pp16 = pp_v[r]
        ppn = one - pp16
        p = [buf_v[slot, 0, pl.ds(t * 16, 16)] * ppn +
             buf_v[slot, 0, pl.ds(64 + t * 16, 16)] * pp16 for t in range(4)]
        for kb in range(NG):
            des_lo = []
            des_hi = []
            for i in range(8):
                k = kb * 8 + i
                if k < K:
                    clo = [buf_v[slot, k, pl.ds(t * 16, 16)]
                           for t in range(4)]
                    chi = [buf_v[slot, k, pl.ds(64 + t * 16, 16)]
                           for t in range(4)]
                    dlo = p[0] * clo[0]
                    nlo = clo[0] * clo[0]
                    dhi = p[0] * chi[0]
                    nhi = chi[0] * chi[0]
                    for t in range(1, 4):
                        dlo = dlo + p[t] * clo[t]
                        nlo = nlo + clo[t] * clo[t]
                        dhi = dhi + p[t] * chi[t]
                        nhi = nhi + chi[t] * chi[t]
                    des_lo.append((dlo, nlo))
                    des_hi.append((dhi, nhi))
                else:
                    des_lo.append((zero, zero))
                    des_hi.append((zero, zero))
            packed = []
            for des in (des_lo, des_hi):
                a = [None] * 16
                for j in range(16):
                    a[_bitrev4(j)] = des[j][0] if j < 8 else des[j - 8][1]
                lvl = a
                for s in (8, 4, 2, 1):
                    lvl = [merge(lvl[2 * i], lvl[2 * i + 1], s)
                           for i in range(len(lvl) // 2)]
                packed.append(lvl[0])
            pvv = pv_v[r, pl.ds(kb * 16, 16)]
            pk_v[r, pl.ds(kb * 16, 16)] = (packed[0] * (one - pvv) +
                                           packed[1] * pvv)

    wait(RPW)
    pltpu.sync_copy(pk_v, pk_hbm.at[pl.ds(wid * RPW, RPW)])


@functools.cache
def _sc_call():
    return functools.partial(
        pl.kernel,
        mesh=plsc.VectorSubcoreMesh(core_axis_name="c", subcore_axis_name="s"),
        out_type=jax.ShapeDtypeStruct((B, PW), jnp.float32),
        scratch_types=[
            pltpu.VMEM((RPW + 1, K), jnp.int32),
            pltpu.VMEM((RPW, 16), jnp.float32),
            pltpu.VMEM((RPW, PW), jnp.float32),
            pltpu.VMEM((2, K, 128), jnp.float32),
            pltpu.VMEM((RPW, PW), jnp.float32),
            pltpu.SemaphoreType.DMA((2,)),
        ],
        compiler_params=pltpu.CompilerParams(use_tc_tiling_on_sc=True),
    )(_sc_kernel)


RB = 4000


def _repack_kernel(w_ref, out_ref):
    out_ref[...] = jnp.concatenate([w_ref[0], w_ref[1]], axis=1)


def _repack(w):
    n = w.shape[0]
    nb = (n // 2) // RB
    w3 = w.reshape(2, n // 2, D)
    return pl.pallas_call(
        _repack_kernel,
        grid=(nb,),
        in_specs=[pl.BlockSpec((2, RB, D), lambda i: (0, i, 0))],
        out_specs=pl.BlockSpec((RB, 2 * D), lambda i: (i, 0)),
        out_shape=jax.ShapeDtypeStruct((n // 2, 2 * D), jnp.float32),
    )(w3)


def _acos(x):
    return jnp.arctan2(jnp.sqrt(jnp.maximum(1.0 - x * x, 0.0)), x)


def _asin(x):
    return jnp.arctan2(x, jnp.sqrt(jnp.maximum(1.0 - x * x, 0.0)))


def _tc_kernel(pk_ref, out_ref):
    pk = pk_ref[...]
    dt = jnp.concatenate(
        [pk[:, kb * 16:kb * 16 + 8] for kb in range(NG)], axis=1)
    n2 = jnp.concatenate(
        [pk[:, kb * 16 + 8:kb * 16 + 16] for kb in range(NG)], axis=1)

    maxnorm = 1.0 - 1e-5
    sinh_cr = math.sinh(RADIUS)

    raw_n = jnp.sqrt(n2)
    n_cl = jnp.maximum(raw_n, MIN_NORM)
    f = jnp.where(n_cl > maxnorm, maxnorm / n_cl, 1.0)
    proj_n = raw_n * f
    f0 = f[:, 0:1]
    dot_p = dt * (f * f0)

    np_a = jnp.maximum(proj_n[:, 0:1], MIN_NORM)
    np2_a = np_a * np_a
    sin_beta = sinh_cr * (1.0 - np2_a) / (2.0 * np_a)
    beta = _asin(jnp.clip(sin_beta, -1.0, 1.0))
    nc_a = jnp.maximum(proj_n, MIN_NORM)
    cos_alpha = dot_p / (np_a * nc_a)
    cos_alpha = jnp.clip(cos_alpha, -1.0 + 1e-7, 1.0 - 1e-7)
    alpha = _acos(cos_alpha)
    denom = nc_a * jnp.sin(beta - alpha) - np_a * sin_beta
    denom = jnp.where(jnp.abs(denom) < 1e-12, -1e-12, denom)
    h = 0.5 * (np2_a - nc_a * nc_a) / denom
    r_x_y = np2_a + h * h + 2.0 * h * np_a * sin_beta
    cos_angle = jnp.where(h < r_x_y,
                          h / r_x_y * jnp.cos(beta),
                          h / r_x_y * jnp.sin(beta))
    angle_ = _acos(jnp.clip(cos_angle, -1.0, 1.0))
    angle = jnp.where(alpha > beta, angle_, alpha)

    np_h = proj_n[:, 0:1]
    np2_h = np_h * np_h
    sin_beta_h = sinh_cr * (1.0 - np2_h) / (2.0 * np_h)
    hx = (1.0 - np2_h) / (2.0 * sin_beta_h)
    cos_theta = (1.0 + np2_h) / (2.0 * jnp.sqrt(hx * hx + np2_h))
    half_ap = jnp.pi / 2.0 - _acos(jnp.clip(cos_theta, -1.0, 1.0))

    out_ref[...] = angle - half_ap


def _tc_call(pk):
    return pl.pallas_call(
        _tc_kernel,
        out_shape=jax.ShapeDtypeStruct((B, NG * 8), jnp.float32),
    )(pk)


def kernel(inputs, weight):
    idx = inputs.astype(jnp.int32)
    hi = idx >= WP
    i2 = jnp.where(hi, idx - WP, idx).reshape(NW, RPW, K)
    par = hi.astype(jnp.float32)
    pp = jnp.broadcast_to(par[:, 0:1], (B, 16)).reshape(NW, RPW, 16)
    grp = jnp.pad(par, ((0, 0), (0, NG * 8 - K)))
    pv = jnp.concatenate(
        [grp.reshape(B, NG, 1, 8)] * 2, axis=2).reshape(NW, RPW, PW)
    w2 = _repack(weight)
    pk = _sc_call()(i2, pp, pv, w2)
    out = _tc_call(pk)
    return out[:, 1:K]
